# Initial kernel scaffold; baseline (speedup 1.0000x reference)
import jax
import jax.numpy as jnp
from jax.experimental import pallas as pl


def kernel(x, table):
    raise NotImplementedError("write your pallas kernel here")


# speedup vs baseline: 1.5482x; 1.5482x over previous
import functools

import jax
import jax.numpy as jnp
from jax import lax
from jax.experimental import pallas as pl
from jax.experimental.pallas import tpu as pltpu
from jax.experimental.pallas import tpu_sc as plsc

_NUM_CORES = 2
_NUM_SUBCORES = 16
_NW = _NUM_CORES * _NUM_SUBCORES
_CHUNK = 32


@functools.lru_cache(maxsize=None)
def _make_gather(v, d, n_chunks, chunk):
    mesh = plsc.VectorSubcoreMesh(core_axis_name="c", subcore_axis_name="s")

    @functools.partial(
        pl.kernel,
        mesh=mesh,
        out_type=jax.ShapeDtypeStruct((_NW * n_chunks * chunk, d), jnp.float32),
        scratch_types=[
            pltpu.VMEM((n_chunks, chunk), jnp.int32),
            pltpu.VMEM((chunk, d), jnp.float32),
            pltpu.VMEM((chunk, d), jnp.float32),
            pltpu.SemaphoreType.DMA,
            pltpu.SemaphoreType.DMA,
            pltpu.SemaphoreType.DMA,
            pltpu.SemaphoreType.DMA,
        ],
    )
    def gather_kernel(idx_hbm, table_hbm, out_hbm, idx_v, rows0, rows1,
                      g0, g1, o0, o1):
        wid = lax.axis_index("s") * _NUM_CORES + lax.axis_index("c")
        base = wid * (n_chunks * chunk)
        pltpu.sync_copy(idx_hbm.at[wid], idx_v)

        rows = (rows0, rows1)
        gsem = (g0, g1)
        osem = (o0, o1)
        gathers = {}
        outs = {}
        for j in range(min(2, n_chunks)):
            gathers[j] = pltpu.async_copy(
                table_hbm.at[idx_v.at[j]], rows[j], gsem[j])
        for j in range(n_chunks):
            b = j % 2
            gathers[j].wait()
            outs[j] = pltpu.async_copy(
                rows[b], out_hbm.at[pl.ds(base + j * chunk, chunk)], osem[b])
            nj = j + 2
            if nj < n_chunks:
                outs[j].wait()
                gathers[nj] = pltpu.async_copy(
                    table_hbm.at[idx_v.at[nj]], rows[b], gsem[b])
        for j in range(max(0, n_chunks - 2), n_chunks):
            outs[j].wait()

    return gather_kernel


def kernel(x, table):
    b, s = x.shape
    v, d = table.shape
    n = b * s
    n_chunks = n // (_NW * _CHUNK)
    idx = x.reshape(_NW, n_chunks, _CHUNK).astype(jnp.int32)
    rows = _make_gather(v, d, n_chunks, _CHUNK)(idx, table)
    return rows.reshape(b, s, d)
